# baseline (speedup 1.0000x reference)
import functools

import jax
import jax.numpy as jnp
from jax import lax
from jax.experimental import pallas as pl
from jax.experimental.pallas import tpu as pltpu
from jax.experimental.pallas import tpu_sc as plsc

D_MODEL = 64
SCALE = 8.0
NBUF = 2


@functools.lru_cache(maxsize=None)
def _make_gather(n_b0: int, n_b1: int):
  info = plsc.get_sparse_core_info()
  nc, ns, nl = info.num_cores, info.num_subcores, info.num_lanes
  nw = nc * ns
  lanes = 8 * nl
  assert n_b0 == nw * lanes
  mesh = plsc.VectorSubcoreMesh(core_axis_name="c", subcore_axis_name="s")

  @functools.partial(
      pl.kernel,
      mesh=mesh,
      compiler_params=pltpu.CompilerParams(
          use_tc_tiling_on_sc=False, needs_layout_passes=False),
      out_type=jax.ShapeDtypeStruct(
          (n_b1, D_MODEL // 8, nw, 8, lanes), jnp.float32),
      scratch_types=[
          pltpu.VMEM((NBUF, lanes), jnp.int32),
          pltpu.VMEM((lanes, D_MODEL), jnp.float32),
          pltpu.VMEM((lanes, D_MODEL), jnp.float32),
          pltpu.VMEM((NBUF, D_MODEL // 8, 8, lanes), jnp.float32),
          pltpu.SemaphoreType.DMA,
          pltpu.SemaphoreType.DMA,
          pltpu.SemaphoreType.DMA,
          pltpu.SemaphoreType.DMA,
          pltpu.SemaphoreType.DMA,
          pltpu.SemaphoreType.DMA,
      ],
  )
  def k(xt_hbm, table_hbm, out_hbm, idx_v, rows_v0, rows_v1, t_v,
        si0, si1, sg0, sg1, sw0, sw1):
    wid = lax.axis_index("s") * nc + lax.axis_index("c")
    col0 = wid * lanes
    sem_i = (si0, si1)
    sem_g = (sg0, sg1)
    sem_w = (sw0, sw1)
    rows_v = (rows_v0, rows_v1)

    def idx_copy(c, b):
      return pltpu.make_async_copy(
          xt_hbm.at[c, pl.ds(col0, lanes)], idx_v.at[b], sem_i[b])

    def gather_copy(b):
      return pltpu.make_async_copy(
          table_hbm.at[idx_v.at[b]], rows_v[b], sem_g[b])

    def out_copy(c, b):
      return pltpu.make_async_copy(
          t_v.at[b], out_hbm.at[c, :, wid], sem_w[b])

    def transpose_scale(b):
      @pl.loop(0, D_MODEL * (lanes // 16), unroll=8)
      def _(t):
        f = t >> 3
        g = t & 7
        rows = g * 16 + lax.iota(jnp.int32, 16)
        cols = jnp.full((16,), 0, jnp.int32) + f
        v = plsc.load_gather(rows_v[b], [rows, cols])
        t_v[b, f >> 3, f & 7, pl.ds(g * 16, 16)] = v * SCALE

    idx_copy(0, 0).start()
    idx_copy(0, 0).wait()
    gather_copy(0).start()
    idx_copy(1, 1).start()

    @pl.loop(0, n_b1, step=NBUF)
    def _(c0):
      for boff in range(NBUF):
        c = c0 + boff
        b = boff
        nb = 1 - boff

        @pl.when(c + 1 < n_b1)
        def _():
          idx_copy(c + 1, nb).wait()
          gather_copy(nb).start()

        @pl.when(c >= NBUF)
        def _():
          out_copy(c - NBUF, b).wait()

        gather_copy(b).wait()

        @pl.when(c + NBUF < n_b1)
        def _():
          idx_copy(c + NBUF, b).start()

        transpose_scale(b)
        out_copy(c, b).start()

    out_copy(n_b1 - NBUF, 0).wait()
    out_copy(n_b1 - 1, 1).wait()

  return k


def kernel(x, table):
  b0, b1 = x.shape
  xt = x.T.astype(jnp.int32)
  out5 = _make_gather(b0, b1)(xt, table)
  return out5.transpose((2, 4, 0, 1, 3)).reshape(b0, b1, D_MODEL)


# speedup vs baseline: 1.1342x; 1.1342x over previous
import functools

import jax
import jax.numpy as jnp
from jax import lax
from jax.experimental import pallas as pl
from jax.experimental.pallas import tpu as pltpu
from jax.experimental.pallas import tpu_sc as plsc

D_MODEL = 64
SCALE = 8.0
NBUF = 2
CB = 2


@functools.lru_cache(maxsize=None)
def _make_gather(n_b0: int, n_b1: int):
  info = plsc.get_sparse_core_info()
  nc, ns, nl = info.num_cores, info.num_subcores, info.num_lanes
  nw = nc * ns
  lanes = 8 * nl
  assert n_b0 == nw * lanes
  chunks = n_b1 // CB
  mesh = plsc.VectorSubcoreMesh(core_axis_name="c", subcore_axis_name="s")

  @functools.partial(
      pl.kernel,
      mesh=mesh,
      compiler_params=pltpu.CompilerParams(
          use_tc_tiling_on_sc=False, needs_layout_passes=False),
      out_type=jax.ShapeDtypeStruct(
          (n_b1, D_MODEL // 8, nw, 8 * lanes), jnp.float32),
      scratch_types=[
          pltpu.VMEM((NBUF, CB, lanes), jnp.int32),
          pltpu.VMEM((NBUF, CB * lanes, D_MODEL), jnp.float32),
          pltpu.VMEM((NBUF, CB, D_MODEL // 8, 8 * lanes), jnp.float32),
          pltpu.SemaphoreType.DMA,
          pltpu.SemaphoreType.DMA,
          pltpu.SemaphoreType.DMA,
          pltpu.SemaphoreType.DMA,
          pltpu.SemaphoreType.DMA,
          pltpu.SemaphoreType.DMA,
      ],
  )
  def k(xt_hbm, table_hbm, out_hbm, idx_v, rows_v, t_v,
        si0, si1, sg0, sg1, sw0, sw1):
    wid = lax.axis_index("s") * nc + lax.axis_index("c")
    col0 = wid * lanes
    sem_i = (si0, si1)
    sem_g = (sg0, sg1)
    sem_w = (sw0, sw1)

    def idx_copy(c, b):
      return pltpu.make_async_copy(
          xt_hbm.at[pl.ds(c * CB, CB), pl.ds(col0, lanes)],
          idx_v.at[b], sem_i[b])

    def gather_copies(b):
      return [
          pltpu.make_async_copy(
              table_hbm.at[idx_v.at[b, cb]],
              rows_v.at[b, pl.ds(cb * lanes, lanes)], sem_g[b])
          for cb in range(CB)
      ]

    def out_copy(c, b):
      return pltpu.make_async_copy(
          t_v.at[b], out_hbm.at[pl.ds(c * CB, CB), :, wid], sem_w[b])

    iot = lax.iota(jnp.int32, 16)
    fi_vecs = [(iot + 16 * kk) >> 3 for kk in range(D_MODEL // 16)]
    in_base = [((iot + 16 * kk) & 7) * 128 for kk in range(D_MODEL // 16)]

    def transpose_scale(b):
      for cb in range(CB):
        tref = t_v.at[b, cb]

        @pl.loop(0, lanes, unroll=4)
        def _(r):
          rvec = jnp.full((16,), 0, jnp.int32) + r
          for kk in range(D_MODEL // 16):
            v = rows_v[b, cb * lanes + r, pl.ds(16 * kk, 16)] * SCALE
            plsc.store_scatter(tref, [fi_vecs[kk], in_base[kk] + rvec], v)

    idx_copy(0, 0).start()
    idx_copy(0, 0).wait()
    for cp in gather_copies(0):
      cp.start()
    idx_copy(1, 1).start()

    @pl.loop(0, chunks, step=NBUF)
    def _(c0):
      for boff in range(NBUF):
        c = c0 + boff
        b = boff
        nb = 1 - boff

        @pl.when(c + 1 < chunks)
        def _():
          idx_copy(c + 1, nb).wait()
          for cp in gather_copies(nb):
            cp.start()

        @pl.when(c >= NBUF)
        def _():
          out_copy(c - NBUF, b).wait()

        for cp in gather_copies(b):
          cp.wait()

        @pl.when(c + NBUF < chunks)
        def _():
          idx_copy(c + NBUF, b).start()

        transpose_scale(b)
        out_copy(c, b).start()

    out_copy(chunks - NBUF, 0).wait()
    out_copy(chunks - 1, 1).wait()

  return k


def kernel(x, table):
  b0, b1 = x.shape
  xt = x.T.astype(jnp.int32)
  out4 = _make_gather(b0, b1)(xt, table)
  out5 = out4.reshape(b1, D_MODEL // 8, 32, 8, 128)
  return out5.transpose((2, 4, 0, 1, 3)).reshape(b0, b1, D_MODEL)


# speedup vs baseline: 1.7301x; 1.5254x over previous
import functools

import jax
import jax.numpy as jnp
from jax import lax
from jax.experimental import pallas as pl
from jax.experimental.pallas import tpu as pltpu
from jax.experimental.pallas import tpu_sc as plsc

D_MODEL = 64
SCALE = 8.0
NBUF = 2
CB = 2


@functools.lru_cache(maxsize=None)
def _make_gather(n_b0: int, n_b1: int):
  info = plsc.get_sparse_core_info()
  nc, ns, nl = info.num_cores, info.num_subcores, info.num_lanes
  nw = nc * ns
  lanes = 8 * nl
  assert n_b0 == nw * lanes
  chunks = n_b1 // CB
  mesh = plsc.VectorSubcoreMesh(core_axis_name="c", subcore_axis_name="s")

  @functools.partial(
      pl.kernel,
      mesh=mesh,
      compiler_params=pltpu.CompilerParams(
          use_tc_tiling_on_sc=False, needs_layout_passes=False),
      out_type=jax.ShapeDtypeStruct(
          (n_b1, D_MODEL // 8, nw, 8, lanes), jnp.float32),
      scratch_types=[
          pltpu.VMEM((NBUF, CB, lanes), jnp.int32),
          pltpu.VMEM((NBUF, CB * lanes, D_MODEL), jnp.float32),
          pltpu.VMEM((NBUF, CB, D_MODEL // 8, 8, lanes + 1), jnp.float32),
          pltpu.SemaphoreType.DMA,
          pltpu.SemaphoreType.DMA,
          pltpu.SemaphoreType.DMA,
          pltpu.SemaphoreType.DMA,
          pltpu.SemaphoreType.DMA,
          pltpu.SemaphoreType.DMA,
      ],
  )
  def k(xt_hbm, table_hbm, out_hbm, idx_v, rows_v, t_v,
        si0, si1, sg0, sg1, sw0, sw1):
    wid = lax.axis_index("s") * nc + lax.axis_index("c")
    col0 = wid * lanes
    sem_i = (si0, si1)
    sem_g = (sg0, sg1)
    sem_w = (sw0, sw1)

    def idx_copy(c, b):
      return pltpu.make_async_copy(
          xt_hbm.at[pl.ds(c * CB, CB), pl.ds(col0, lanes)],
          idx_v.at[b], sem_i[b])

    def gather_copies(b):
      return [
          pltpu.make_async_copy(
              table_hbm.at[idx_v.at[b, cb]],
              rows_v.at[b, pl.ds(cb * lanes, lanes)], sem_g[b])
          for cb in range(CB)
      ]

    def out_copy(c, b):
      return pltpu.make_async_copy(
          t_v.at[b, :, :, :, pl.ds(0, lanes)],
          out_hbm.at[pl.ds(c * CB, CB), :, wid], sem_w[b])

    iot = lax.iota(jnp.int32, 16)
    fi_vecs = [(iot + 16 * kk) >> 3 for kk in range(D_MODEL // 16)]
    fs_vecs = [(iot + 16 * kk) & 7 for kk in range(D_MODEL // 16)]

    def transpose_scale(b):
      for cb in range(CB):
        tref = t_v.at[b, cb]

        @pl.loop(0, lanes, unroll=4)
        def _(r):
          rvec = jnp.full((16,), 0, jnp.int32) + r
          for kk in range(D_MODEL // 16):
            v = rows_v[b, cb * lanes + r, pl.ds(16 * kk, 16)] * SCALE
            plsc.store_scatter(tref, [fi_vecs[kk], fs_vecs[kk], rvec], v)

    idx_copy(0, 0).start()
    idx_copy(0, 0).wait()
    for cp in gather_copies(0):
      cp.start()
    idx_copy(1, 1).start()

    @pl.loop(0, chunks, step=NBUF)
    def _(c0):
      for boff in range(NBUF):
        c = c0 + boff
        b = boff
        nb = 1 - boff

        @pl.when(c + 1 < chunks)
        def _():
          idx_copy(c + 1, nb).wait()
          for cp in gather_copies(nb):
            cp.start()

        @pl.when(c >= NBUF)
        def _():
          out_copy(c - NBUF, b).wait()

        for cp in gather_copies(b):
          cp.wait()

        @pl.when(c + NBUF < chunks)
        def _():
          idx_copy(c + NBUF, b).start()

        transpose_scale(b)
        out_copy(c, b).start()

    out_copy(chunks - NBUF, 0).wait()
    out_copy(chunks - 1, 1).wait()

  return k


def kernel(x, table):
  b0, b1 = x.shape
  xt = x.T.astype(jnp.int32)
  out5 = _make_gather(b0, b1)(xt, table)
  return out5.transpose((2, 4, 0, 1, 3)).reshape(b0, b1, D_MODEL)


# speedup vs baseline: 2.5688x; 1.4848x over previous
import functools

import jax
import jax.numpy as jnp
from jax import lax
from jax.experimental import pallas as pl
from jax.experimental.pallas import tpu as pltpu
from jax.experimental.pallas import tpu_sc as plsc

D_MODEL = 64
SCALE = 8.0
NBUF = 2
CB = 2


@functools.lru_cache(maxsize=None)
def _make_gather(n_b0: int, n_b1: int):
  info = plsc.get_sparse_core_info()
  nc, ns, nl = info.num_cores, info.num_subcores, info.num_lanes
  nw = nc * ns
  lanes = 8 * nl
  assert n_b0 == nw * lanes
  chunks = n_b1 // CB
  mesh = plsc.VectorSubcoreMesh(core_axis_name="c", subcore_axis_name="s")

  @functools.partial(
      pl.kernel,
      mesh=mesh,
      compiler_params=pltpu.CompilerParams(
          use_tc_tiling_on_sc=False, needs_layout_passes=False),
      out_type=jax.ShapeDtypeStruct(
          (n_b1, D_MODEL // 8, nw, 8, lanes), jnp.float32),
      scratch_types=[
          pltpu.VMEM((NBUF, CB, lanes), jnp.int32),
          pltpu.VMEM((NBUF, CB * lanes, D_MODEL), jnp.float32),
          pltpu.VMEM((NBUF, CB, D_MODEL // 8, 8, lanes + 1), jnp.float32),
          pltpu.SemaphoreType.DMA,
          pltpu.SemaphoreType.DMA,
          pltpu.SemaphoreType.DMA,
          pltpu.SemaphoreType.DMA,
          pltpu.SemaphoreType.DMA,
          pltpu.SemaphoreType.DMA,
      ],
  )
  def k(xt_hbm, table_hbm, out_hbm, idx_v, rows_v, t_v,
        si0, si1, sg0, sg1, sw0, sw1):
    wid = lax.axis_index("s") * nc + lax.axis_index("c")
    col0 = wid * lanes
    sem_i = (si0, si1)
    sem_g = (sg0, sg1)
    sem_w = (sw0, sw1)

    def idx_copy(c, b):
      return pltpu.make_async_copy(
          xt_hbm.at[pl.ds(c * CB, CB), pl.ds(col0, lanes)],
          idx_v.at[b], sem_i[b])

    def gather_copies(b):
      return [
          pltpu.make_async_copy(
              table_hbm.at[idx_v.at[b, cb]],
              rows_v.at[b, pl.ds(cb * lanes, lanes)], sem_g[b])
          for cb in range(CB)
      ]

    def out_copy(c, b):
      return pltpu.make_async_copy(
          t_v.at[b, :, :, :, pl.ds(0, lanes)],
          out_hbm.at[pl.ds(c * CB, CB), :, wid], sem_w[b])

    iot = lax.iota(jnp.int32, 16)
    fi_vecs = [(iot + 16 * kk) >> 3 for kk in range(D_MODEL // 16)]
    fs_vecs = [(iot + 16 * kk) & 7 for kk in range(D_MODEL // 16)]

    def transpose_scale(b):
      for cb in range(CB):
        tref = t_v.at[b, cb]

        @plsc.parallel_loop(0, lanes, unroll=4)
        def _(r):
          rvec = jnp.full((16,), 0, jnp.int32) + r
          for kk in range(D_MODEL // 16):
            v = rows_v[b, cb * lanes + r, pl.ds(16 * kk, 16)] * SCALE
            plsc.store_scatter(tref, [fi_vecs[kk], fs_vecs[kk], rvec], v)

    idx_copy(0, 0).start()
    idx_copy(0, 0).wait()
    for cp in gather_copies(0):
      cp.start()
    idx_copy(1, 1).start()

    @pl.loop(0, chunks, step=NBUF)
    def _(c0):
      for boff in range(NBUF):
        c = c0 + boff
        b = boff
        nb = 1 - boff

        @pl.when(c + 1 < chunks)
        def _():
          idx_copy(c + 1, nb).wait()
          for cp in gather_copies(nb):
            cp.start()

        @pl.when(c >= NBUF)
        def _():
          out_copy(c - NBUF, b).wait()

        for cp in gather_copies(b):
          cp.wait()

        @pl.when(c + NBUF < chunks)
        def _():
          idx_copy(c + NBUF, b).start()

        transpose_scale(b)
        out_copy(c, b).start()

    out_copy(chunks - NBUF, 0).wait()
    out_copy(chunks - 1, 1).wait()

  return k


def kernel(x, table):
  b0, b1 = x.shape
  xt = x.T.astype(jnp.int32)
  out5 = _make_gather(b0, b1)(xt, table)
  return out5.transpose((2, 4, 0, 1, 3)).reshape(b0, b1, D_MODEL)
